# baseline (speedup 1.0000x reference)
import jax
import jax.numpy as jnp
from jax import lax
from jax.experimental import pallas as pl
from jax.experimental.pallas import tpu as pltpu
from jax.experimental.pallas import tpu_sc as plsc

_B = 16384
_D = 64
_L = 16
_NC = 2
_NS = 16
_NW = _NC * _NS
_BPW = _B // _NW
_NG = _BPW // _L


def _bpr_body(ubase_hbm, ibase_hbm, usub_hbm, isub_hbm, uemb_hbm, iemb_hbm,
              out_hbm, ubase_v, ibase_v, usub_v, isub_v, ublk_v, iblk_v,
              out_v, sem):
    wid = lax.axis_index("s") * _NC + lax.axis_index("c")
    base = wid * _BPW

    pltpu.sync_copy(ubase_hbm.at[wid], ubase_v)
    pltpu.sync_copy(ibase_hbm.at[wid], ibase_v)
    pltpu.sync_copy(usub_hbm.at[wid], usub_v)
    pltpu.sync_copy(isub_hbm.at[wid], isub_v)

    lane = lax.iota(jnp.int32, _L)
    gat_dnums = lax.GatherDimensionNumbers(
        offset_dims=(), collapsed_slice_dims=(0,), start_index_map=(0,))
    rot_idx = [jnp.bitwise_and(lane + sh, _L - 1) for sh in (8, 4, 2, 1)]

    def _lane_rotate(p, idx):
        return lax.gather(p, idx[:, None], gat_dnums, (1,),
                          mode=lax.GatherScatterMode.PROMISE_IN_BOUNDS)

    def group(g, carry):
        ubv = ubase_v[pl.ds(g * _L, _L)]
        ibv = ibase_v[pl.ds(g * _L, _L)]
        copies = []
        for k in range(_L):
            copies.append(pltpu.async_copy(
                uemb_hbm.at[pl.ds(pl.multiple_of(ubv[k], 8), 8)],
                ublk_v.at[k], sem))
            copies.append(pltpu.async_copy(
                iemb_hbm.at[pl.ds(pl.multiple_of(ibv[k], 8), 8)],
                iblk_v.at[k], sem))
        for c in copies:
            c.wait()

        suv = usub_v[pl.ds(g * _L, _L)]
        siv = isub_v[pl.ds(g * _L, _L)]
        dots = jnp.zeros((_L,), jnp.float32)
        for k in range(_L):
            su = suv[k]
            si = siv[k]
            p = ublk_v[k, su, pl.ds(0, _L)] * iblk_v[k, si, pl.ds(0, _L)]
            for c in range(1, _D // _L):
                p = p + (ublk_v[k, su, pl.ds(c * _L, _L)]
                         * iblk_v[k, si, pl.ds(c * _L, _L)])
            for idx in rot_idx:
                p = p + _lane_rotate(p, idx)
            dots = jnp.where(lane == k, p, dots)
        out_v[pl.ds(g * _L, _L)] = dots
        return carry

    lax.fori_loop(0, _NG, group, 0)
    pltpu.sync_copy(out_v, out_hbm.at[pl.ds(base, _BPW)])


def kernel(users, items, user_emb, item_emb):
    users = users.astype(jnp.int32)
    items = items.astype(jnp.int32)
    ubase = (users & ~7).reshape(_NW, _BPW)
    ibase = (items & ~7).reshape(_NW, _BPW)
    usub = (users & 7).reshape(_NW, _BPW)
    isub = (items & 7).reshape(_NW, _BPW)
    mesh = plsc.VectorSubcoreMesh(core_axis_name="c", subcore_axis_name="s")
    run = pl.kernel(
        _bpr_body,
        out_type=jax.ShapeDtypeStruct((_B,), jnp.float32),
        mesh=mesh,
        scratch_types=[
            pltpu.VMEM((_BPW,), jnp.int32),
            pltpu.VMEM((_BPW,), jnp.int32),
            pltpu.VMEM((_BPW,), jnp.int32),
            pltpu.VMEM((_BPW,), jnp.int32),
            pltpu.VMEM((_L, 8, _D), jnp.float32),
            pltpu.VMEM((_L, 8, _D), jnp.float32),
            pltpu.VMEM((_BPW,), jnp.float32),
            pltpu.SemaphoreType.DMA,
        ],
    )
    return run(ubase, ibase, usub, isub, user_emb, item_emb)


# speedup vs baseline: 1.0004x; 1.0004x over previous
import jax
import jax.numpy as jnp
from jax import lax
from jax.experimental import pallas as pl
from jax.experimental.pallas import tpu as pltpu
from jax.experimental.pallas import tpu_sc as plsc

_B = 16384
_D = 64
_L = 16
_NC = 2
_NS = 16
_NW = _NC * _NS
_BPW = _B // _NW
_NG = _BPW // _L


def _bpr_body(ubase_hbm, ibase_hbm, usub_hbm, isub_hbm, uemb_hbm, iemb_hbm,
              out_hbm, ubase_v, ibase_v, usub_v, isub_v, ublk_v, iblk_v,
              out_v, sem):
    wid = lax.axis_index("s") * _NC + lax.axis_index("c")
    base = wid * _BPW

    pltpu.sync_copy(ubase_hbm.at[wid], ubase_v)
    pltpu.sync_copy(ibase_hbm.at[wid], ibase_v)
    pltpu.sync_copy(usub_hbm.at[wid], usub_v)
    pltpu.sync_copy(isub_hbm.at[wid], isub_v)

    lane = lax.iota(jnp.int32, _L)
    gat_dnums = lax.GatherDimensionNumbers(
        offset_dims=(), collapsed_slice_dims=(0,), start_index_map=(0,))
    rot_idx = [jnp.bitwise_and(lane + sh, _L - 1) for sh in (8, 4, 2, 1)]

    def _lane_rotate(p, idx):
        return lax.gather(p, idx[:, None], gat_dnums, (1,),
                          mode=lax.GatherScatterMode.PROMISE_IN_BOUNDS)

    def group(g, carry):
        ubv = ubase_v[pl.ds(g * _L, _L)]
        ibv = ibase_v[pl.ds(g * _L, _L)]
        copies = []
        for k in range(_L):
            copies.append(pltpu.async_copy(
                uemb_hbm.at[pl.ds(pl.multiple_of(ubv[k], 8), 8)],
                ublk_v.at[pl.ds(k * 8, 8)], sem))
            copies.append(pltpu.async_copy(
                iemb_hbm.at[pl.ds(pl.multiple_of(ibv[k], 8), 8)],
                iblk_v.at[pl.ds(k * 8, 8)], sem))
        for c in copies:
            c.wait()

        suv = usub_v[pl.ds(g * _L, _L)]
        siv = isub_v[pl.ds(g * _L, _L)]
        dots = jnp.zeros((_L,), jnp.float32)
        for k in range(_L):
            su = suv[k]
            si = siv[k]
            p = ublk_v[k * 8 + su, pl.ds(0, _L)] * iblk_v[k * 8 + si, pl.ds(0, _L)]
            for c in range(1, _D // _L):
                p = p + (ublk_v[k * 8 + su, pl.ds(c * _L, _L)]
                         * iblk_v[k * 8 + si, pl.ds(c * _L, _L)])
            for idx in rot_idx:
                p = p + _lane_rotate(p, idx)
            dots = jnp.where(lane == k, p, dots)
        out_v[pl.ds(g * _L, _L)] = dots
        return carry

    lax.fori_loop(0, _NG, group, 0)
    pltpu.sync_copy(out_v, out_hbm.at[pl.ds(base, _BPW)])


def kernel(users, items, user_emb, item_emb):
    users = users.astype(jnp.int32)
    items = items.astype(jnp.int32)
    ubase = (users & ~7).reshape(_NW, _BPW)
    ibase = (items & ~7).reshape(_NW, _BPW)
    usub = (users & 7).reshape(_NW, _BPW)
    isub = (items & 7).reshape(_NW, _BPW)
    mesh = plsc.VectorSubcoreMesh(core_axis_name="c", subcore_axis_name="s")
    run = pl.kernel(
        _bpr_body,
        out_type=jax.ShapeDtypeStruct((_B,), jnp.float32),
        mesh=mesh,
        scratch_types=[
            pltpu.VMEM((_BPW,), jnp.int32),
            pltpu.VMEM((_BPW,), jnp.int32),
            pltpu.VMEM((_BPW,), jnp.int32),
            pltpu.VMEM((_BPW,), jnp.int32),
            pltpu.VMEM((_L * 8, _D), jnp.float32),
            pltpu.VMEM((_L * 8, _D), jnp.float32),
            pltpu.VMEM((_BPW,), jnp.float32),
            pltpu.SemaphoreType.DMA,
        ],
    )
    return run(ubase, ibase, usub, isub, user_emb, item_emb)
